# baseline (speedup 1.0000x reference)
import jax
import jax.numpy as jnp
from jax.experimental import pallas as pl

_MARGIN = 1.35
_BR = 8


def _scale_body(lab_ref, x_ref, o_ref):
    x = x_ref[...]
    cols = jax.lax.broadcasted_iota(jnp.int32, x.shape, 1)
    o_ref[...] = jnp.where(cols == lab_ref[...], x * _MARGIN, x)


def kernel(logits, label):
    b, v = logits.shape
    lab = label.astype(jnp.int32).reshape(b, 1)
    grid = (b // _BR,)
    return pl.pallas_call(
        _scale_body,
        grid=grid,
        in_specs=[
            pl.BlockSpec((_BR, 1), lambda i: (i, 0)),
            pl.BlockSpec((_BR, v), lambda i: (i, 0)),
        ],
        out_specs=pl.BlockSpec((_BR, v), lambda i: (i, 0)),
        out_shape=jax.ShapeDtypeStruct((b, v), logits.dtype),
    )(lab, logits)


# speedup vs baseline: 1.0242x; 1.0242x over previous
import jax
import jax.numpy as jnp
from jax.experimental import pallas as pl
from jax.experimental.pallas import tpu as pltpu

_MARGIN = 1.35
_RB = 8
_NBUF = 4


def _body(lab_ref, x_hbm, o_hbm, ibuf, obuf, isems, osems):
    n_units = x_hbm.shape[0] // _RB

    def start_in(u, slot):
        pltpu.make_async_copy(
            x_hbm.at[pl.ds(u * _RB, _RB)], ibuf.at[slot], isems.at[slot]
        ).start()

    for s in range(min(_NBUF, n_units)):
        start_in(s, s)

    for u in range(n_units):
        slot = u % _NBUF
        pltpu.make_async_copy(
            x_hbm.at[pl.ds(u * _RB, _RB)], ibuf.at[slot], isems.at[slot]
        ).wait()
        if u >= _NBUF:
            pltpu.make_async_copy(
                obuf.at[slot],
                o_hbm.at[pl.ds((u - _NBUF) * _RB, _RB)],
                osems.at[slot],
            ).wait()
        x = ibuf[slot]
        cols = jax.lax.broadcasted_iota(jnp.int32, x.shape, 1)
        lab = lab_ref[pl.ds(u * _RB, _RB), :]
        obuf[slot] = jnp.where(cols == lab, x * _MARGIN, x)
        pltpu.make_async_copy(
            obuf.at[slot], o_hbm.at[pl.ds(u * _RB, _RB)], osems.at[slot]
        ).start()
        nxt = u + _NBUF
        if nxt < n_units:
            start_in(nxt, slot)

    for u in range(max(0, n_units - _NBUF), n_units):
        slot = u % _NBUF
        pltpu.make_async_copy(
            obuf.at[slot], o_hbm.at[pl.ds(u * _RB, _RB)], osems.at[slot]
        ).wait()


def kernel(logits, label):
    b, v = logits.shape
    lab = label.astype(jnp.int32).reshape(b, 1)
    return pl.pallas_call(
        _body,
        in_specs=[
            pl.BlockSpec(memory_space=pltpu.MemorySpace.VMEM),
            pl.BlockSpec(memory_space=pltpu.MemorySpace.HBM),
        ],
        out_specs=pl.BlockSpec(memory_space=pltpu.MemorySpace.HBM),
        out_shape=jax.ShapeDtypeStruct((b, v), logits.dtype),
        scratch_shapes=[
            pltpu.VMEM((_NBUF, _RB, v), logits.dtype),
            pltpu.VMEM((_NBUF, _RB, v), logits.dtype),
            pltpu.SemaphoreType.DMA((_NBUF,)),
            pltpu.SemaphoreType.DMA((_NBUF,)),
        ],
    )(lab, logits)


# speedup vs baseline: 3.8002x; 3.7105x over previous
import jax
import jax.numpy as jnp
from jax.experimental import pallas as pl

_MARGIN = 1.35
_VB = 8192


def _scale_body(lab_ref, x_ref, o_ref):
    i = pl.program_id(0)
    x = x_ref[...]
    rows = jax.lax.broadcasted_iota(jnp.int32, x.shape, 0) + i * _VB
    o_ref[...] = jnp.where(rows == lab_ref[...], x * _MARGIN, x)


def kernel(logits, label):
    b, v = logits.shape
    xt = logits.T
    lab = label.astype(jnp.int32).reshape(1, b)
    out_t = pl.pallas_call(
        _scale_body,
        grid=(pl.cdiv(v, _VB),),
        in_specs=[
            pl.BlockSpec((1, b), lambda i: (0, 0)),
            pl.BlockSpec((_VB, b), lambda i: (i, 0)),
        ],
        out_specs=pl.BlockSpec((_VB, b), lambda i: (i, 0)),
        out_shape=jax.ShapeDtypeStruct((v, b), logits.dtype),
    )(lab, xt)
    return out_t.T
